# baseline (device time: 148692 ns/iter reference)
import jax
import jax.numpy as jnp
from jax import lax
from jax.experimental import pallas as pl
from jax.experimental.pallas import tpu as pltpu

jax.config.update("jax_compilation_cache_dir", "/tmp/jax_cache")
jax.config.update("jax_persistent_cache_min_compile_time_secs", 0)

N_DEV = 4

_sem_signal = getattr(pl, "semaphore_signal", None) or pltpu.semaphore_signal
_sem_wait = getattr(pl, "semaphore_wait", None) or pltpu.semaphore_wait
_DevIdType = getattr(pl, "DeviceIdType", None) or pltpu.DeviceIdType


def kernel(x, w_mat, scale_x, scale_w):
    m, k_per = x.shape
    k_per2, n = w_mat.shape
    assert k_per == k_per2, (x.shape, w_mat.shape)
    kh = k_per // 2
    kq = k_per // 4
    mc = m // 4


    def body(x_ref, w_ref, sx_ref, sw_ref, out_ref,
             qx, qw, stx, stw, gx_l, gw_l, gx_r, gw_r, gx2, gw2,
             ssems, rsems, stsems):
        me = lax.axis_index("i")
        left = lax.rem(me + N_DEV - 1, N_DEV)
        right = lax.rem(me + 1, N_DEV)

        barrier_sem = pltpu.get_barrier_semaphore()
        for nbr in (left, right):
            _sem_signal(barrier_sem, inc=1, device_id=(nbr,),
                        device_id_type=_DevIdType.MESH)
        stage = [
            pltpu.make_async_copy(
                x_ref.at[pl.ds(c * mc, mc), :], stx.at[c], stsems.at[c])
            for c in range(2)
        ]
        stage_w = pltpu.make_async_copy(w_ref, stw, stsems.at[2])
        stage[0].start()
        stage[1].start()
        stage_w.start()
        _sem_wait(barrier_sem, 2)

        def rc(src, dst, dev, i):
            return pltpu.make_async_remote_copy(
                src_ref=src, dst_ref=dst,
                send_sem=ssems.at[i], recv_sem=rsems.at[i],
                device_id=(dev,), device_id_type=_DevIdType.MESH)

        x_cw, x_ccw = [], []
        for c in range(4):
            stage[c].wait()
            slot = c % 2
            rows = pl.ds(c * mc, mc)
            qx[rows, :] = stx[slot].astype(jnp.float8_e4m3fn)
            cw = rc(qx.at[rows, :], gx_l.at[rows, :], right, c)
            ccw = rc(qx.at[rows, :], gx_r.at[rows, :], left, 4 + c)
            cw.start()
            ccw.start()
            x_cw.append(cw)
            x_ccw.append(ccw)
            if c + 2 < 4:
                nxt = pltpu.make_async_copy(
                    x_ref.at[pl.ds((c + 2) * mc, mc), :], stx.at[slot],
                    stsems.at[slot])
                nxt.start()
                stage.append(nxt)
            if c == 0:
                stage_w.wait()
                qw[...] = stw[...].astype(jnp.float8_e4m3fn)
                w_cw = rc(qw, gw_l, right, 8)
                w_ccw = rc(qw, gw_r, left, 9)
                w_cw.start()
                w_ccw.start()

        out_ref[...] = jnp.dot(
            qx[...], qw[...],
            preferred_element_type=jnp.float32).astype(jnp.bfloat16)

        w_cw.wait()
        w_ccw.wait()
        for c in range(4):
            rows = pl.ds(c * mc, mc)
            x_cw[c].wait()
            out_ref[rows, :] += jnp.dot(
                gx_l[rows, :], gw_l[...],
                preferred_element_type=jnp.float32).astype(jnp.bfloat16)
            x_ccw[c].wait()
            out_ref[rows, :] += jnp.dot(
                gx_r[rows, :], gw_r[...],
                preferred_element_type=jnp.float32).astype(jnp.bfloat16)

        offs = (0, kq, kh, kh + kq)
        hop2 = []
        for c, o in enumerate(offs):
            dev = right if c < 2 else left
            src_x, src_w = (gx_l, gw_l) if c < 2 else (gx_r, gw_r)
            hop2.append((
                rc(src_x.at[:, pl.ds(o, kq)], gx2.at[:, pl.ds(o, kq)],
                   dev, 10 + 2 * c),
                rc(src_w.at[pl.ds(o, kq), :], gw2.at[pl.ds(o, kq), :],
                   dev, 11 + 2 * c),
            ))
        for rx, rw in hop2:
            rx.start()
            rw.start()

        for c in (0, 2, 1, 3):
            rx, rw = hop2[c]
            rx.wait()
            rw.wait()
            o = offs[c]
            out_ref[...] += jnp.dot(
                gx2[:, pl.ds(o, kq)], gw2[pl.ds(o, kq), :],
                preferred_element_type=jnp.float32).astype(jnp.bfloat16)

        y = out_ref[...].astype(jnp.float32) * (sx_ref[0] * sw_ref[0])
        z = jnp.clip(y, -60.0, 60.0)
        out_ref[...] = (y / (1.0 + jnp.exp(-z))).astype(jnp.bfloat16)

    out = pl.pallas_call(
        body,
        out_shape=jax.ShapeDtypeStruct((m, n), jnp.bfloat16),
        in_specs=[
            pl.BlockSpec(memory_space=pl.ANY),
            pl.BlockSpec(memory_space=pl.ANY),
            pl.BlockSpec(memory_space=pltpu.SMEM),
            pl.BlockSpec(memory_space=pltpu.SMEM),
        ],
        out_specs=pl.BlockSpec(memory_space=pltpu.VMEM),
        scratch_shapes=[
            pltpu.VMEM((m, k_per), jnp.float8_e4m3fn),
            pltpu.VMEM((k_per, n), jnp.float8_e4m3fn),
            pltpu.VMEM((2, mc, k_per), jnp.float32),
            pltpu.VMEM((k_per, n), jnp.float32),
            pltpu.VMEM((m, k_per), jnp.float8_e4m3fn),
            pltpu.VMEM((k_per, n), jnp.float8_e4m3fn),
            pltpu.VMEM((m, k_per), jnp.float8_e4m3fn),
            pltpu.VMEM((k_per, n), jnp.float8_e4m3fn),
            pltpu.VMEM((m, k_per), jnp.float8_e4m3fn),
            pltpu.VMEM((k_per, n), jnp.float8_e4m3fn),
            pltpu.SemaphoreType.DMA((18,)),
            pltpu.SemaphoreType.DMA((18,)),
            pltpu.SemaphoreType.DMA((3,)),
        ],
        compiler_params=pltpu.CompilerParams(
            collective_id=0, vmem_limit_bytes=100 * 1024 * 1024),
    )(x, w_mat, scale_x, scale_w)
    return out


# device time: 143789 ns/iter; 1.0341x vs baseline; 1.0341x over previous
import jax
import jax.numpy as jnp
from jax import lax
from jax.experimental import pallas as pl
from jax.experimental.pallas import tpu as pltpu

jax.config.update("jax_compilation_cache_dir", "/tmp/jax_cache")
jax.config.update("jax_persistent_cache_min_compile_time_secs", 0)

N_DEV = 4

_sem_signal = getattr(pl, "semaphore_signal", None) or pltpu.semaphore_signal
_sem_wait = getattr(pl, "semaphore_wait", None) or pltpu.semaphore_wait
_DevIdType = getattr(pl, "DeviceIdType", None) or pltpu.DeviceIdType


def kernel(x, w_mat, scale_x, scale_w):
    m, k_per = x.shape
    k_per2, n = w_mat.shape
    assert k_per == k_per2, (x.shape, w_mat.shape)
    kh = k_per // 2
    kq = k_per // 4
    mc = m // 4


    def body(x_ref, w_ref, sx_ref, sw_ref, out_ref,
             qx, qw, stx, stw, gx_l, gw_l, gx_r, gw_r, gx2, gw2,
             ssems, rsems, stsems):
        me = lax.axis_index("i")
        left = lax.rem(me + N_DEV - 1, N_DEV)
        right = lax.rem(me + 1, N_DEV)

        barrier_sem = pltpu.get_barrier_semaphore()
        for nbr in (left, right):
            _sem_signal(barrier_sem, inc=1, device_id=(nbr,),
                        device_id_type=_DevIdType.MESH)
        stage = [
            pltpu.make_async_copy(
                x_ref.at[pl.ds(c * mc, mc), :], stx.at[c], stsems.at[c])
            for c in range(2)
        ]
        stage_w = pltpu.make_async_copy(w_ref, stw, stsems.at[2])
        stage[0].start()
        stage[1].start()
        stage_w.start()
        _sem_wait(barrier_sem, 2)

        def rc(src, dst, dev, i):
            return pltpu.make_async_remote_copy(
                src_ref=src, dst_ref=dst,
                send_sem=ssems.at[i], recv_sem=rsems.at[i],
                device_id=(dev,), device_id_type=_DevIdType.MESH)

        hop1 = []
        for c in range(4):
            stage[c].wait()
            slot = c % 2
            rows = pl.ds(c * mc, mc)
            qx[rows, :] = stx[slot].astype(jnp.float8_e4m3fn)
            cw = rc(qx.at[rows, :], gx_l.at[rows, :], right, c)
            ccw = rc(qx.at[rows, :], gx_r.at[rows, :], left, 4 + c)
            cw.start()
            ccw.start()
            hop1 += [cw, ccw]
            if c + 2 < 4:
                nxt = pltpu.make_async_copy(
                    x_ref.at[pl.ds((c + 2) * mc, mc), :], stx.at[slot],
                    stsems.at[slot])
                nxt.start()
                stage.append(nxt)
            if c == 0:
                stage_w.wait()
                qw[...] = stw[...].astype(jnp.float8_e4m3fn)
                w_cw = rc(qw, gw_l, right, 8)
                w_ccw = rc(qw, gw_r, left, 9)
                w_cw.start()
                w_ccw.start()
                hop1 += [w_cw, w_ccw]

        out_ref[...] = jnp.dot(
            qx[...], qw[...],
            preferred_element_type=jnp.float32).astype(jnp.bfloat16)

        for r in hop1:
            r.wait()

        offs = (0, kq, kh, kh + kq)
        hop2 = []
        for c, o in enumerate(offs):
            dev = right if c < 2 else left
            src_x, src_w = (gx_l, gw_l) if c < 2 else (gx_r, gw_r)
            hop2.append((
                rc(src_x.at[:, pl.ds(o, kq)], gx2.at[:, pl.ds(o, kq)],
                   dev, 10 + 2 * c),
                rc(src_w.at[pl.ds(o, kq), :], gw2.at[pl.ds(o, kq), :],
                   dev, 11 + 2 * c),
            ))
        for rx, rw in hop2:
            rx.start()
            rw.start()

        out_ref[...] += jnp.dot(
            gx_l[...], gw_l[...],
            preferred_element_type=jnp.float32).astype(jnp.bfloat16)
        out_ref[...] += jnp.dot(
            gx_r[...], gw_r[...],
            preferred_element_type=jnp.float32).astype(jnp.bfloat16)

        for c in (0, 2, 1, 3):
            rx, rw = hop2[c]
            rx.wait()
            rw.wait()
            o = offs[c]
            out_ref[...] += jnp.dot(
                gx2[:, pl.ds(o, kq)], gw2[pl.ds(o, kq), :],
                preferred_element_type=jnp.float32).astype(jnp.bfloat16)

        y = out_ref[...].astype(jnp.float32) * (sx_ref[0] * sw_ref[0])
        z = jnp.clip(y, -60.0, 60.0)
        out_ref[...] = (y / (1.0 + jnp.exp(-z))).astype(jnp.bfloat16)

    out = pl.pallas_call(
        body,
        out_shape=jax.ShapeDtypeStruct((m, n), jnp.bfloat16),
        in_specs=[
            pl.BlockSpec(memory_space=pl.ANY),
            pl.BlockSpec(memory_space=pl.ANY),
            pl.BlockSpec(memory_space=pltpu.SMEM),
            pl.BlockSpec(memory_space=pltpu.SMEM),
        ],
        out_specs=pl.BlockSpec(memory_space=pltpu.VMEM),
        scratch_shapes=[
            pltpu.VMEM((m, k_per), jnp.float8_e4m3fn),
            pltpu.VMEM((k_per, n), jnp.float8_e4m3fn),
            pltpu.VMEM((2, mc, k_per), jnp.float32),
            pltpu.VMEM((k_per, n), jnp.float32),
            pltpu.VMEM((m, k_per), jnp.float8_e4m3fn),
            pltpu.VMEM((k_per, n), jnp.float8_e4m3fn),
            pltpu.VMEM((m, k_per), jnp.float8_e4m3fn),
            pltpu.VMEM((k_per, n), jnp.float8_e4m3fn),
            pltpu.VMEM((m, k_per), jnp.float8_e4m3fn),
            pltpu.VMEM((k_per, n), jnp.float8_e4m3fn),
            pltpu.SemaphoreType.DMA((18,)),
            pltpu.SemaphoreType.DMA((18,)),
            pltpu.SemaphoreType.DMA((3,)),
        ],
        compiler_params=pltpu.CompilerParams(
            collective_id=0, vmem_limit_bytes=100 * 1024 * 1024),
    )(x, w_mat, scale_x, scale_w)
    return out


# device time: 141611 ns/iter; 1.0500x vs baseline; 1.0154x over previous
import jax
import jax.numpy as jnp
from jax import lax
from jax.experimental import pallas as pl
from jax.experimental.pallas import tpu as pltpu

jax.config.update("jax_compilation_cache_dir", "/tmp/jax_cache")
jax.config.update("jax_persistent_cache_min_compile_time_secs", 0)

N_DEV = 4

_sem_signal = getattr(pl, "semaphore_signal", None) or pltpu.semaphore_signal
_sem_wait = getattr(pl, "semaphore_wait", None) or pltpu.semaphore_wait
_DevIdType = getattr(pl, "DeviceIdType", None) or pltpu.DeviceIdType


def kernel(x, w_mat, scale_x, scale_w):
    m, k_per = x.shape
    k_per2, n = w_mat.shape
    assert k_per == k_per2, (x.shape, w_mat.shape)
    kh = k_per // 2
    kq = k_per // 4
    mc = m // 4


    def body(x_ref, w_ref, sx_ref, sw_ref, out_ref,
             qx, qw, stx, stw, gx_l, gw_l, gx_r, gw_r, gx2, gw2,
             ssems, rsems, stsems):
        me = lax.axis_index("i")
        left = lax.rem(me + N_DEV - 1, N_DEV)
        right = lax.rem(me + 1, N_DEV)

        barrier_sem = pltpu.get_barrier_semaphore()
        for nbr in (left, right):
            _sem_signal(barrier_sem, inc=1, device_id=(nbr,),
                        device_id_type=_DevIdType.MESH)
        stage = [
            pltpu.make_async_copy(
                x_ref.at[pl.ds(c * mc, mc), :], stx.at[c], stsems.at[c])
            for c in range(2)
        ]
        stage_w = pltpu.make_async_copy(w_ref, stw, stsems.at[2])
        stage[0].start()
        stage[1].start()
        stage_w.start()
        _sem_wait(barrier_sem, 2)

        def rc(src, dst, dev, i):
            return pltpu.make_async_remote_copy(
                src_ref=src, dst_ref=dst,
                send_sem=ssems.at[i], recv_sem=rsems.at[i],
                device_id=(dev,), device_id_type=_DevIdType.MESH)

        hop1 = []
        for c in range(4):
            stage[c].wait()
            slot = c % 2
            rows = pl.ds(c * mc, mc)
            qx[rows, :] = stx[slot].astype(jnp.float8_e4m3fn)
            cw = rc(qx.at[rows, :], gx_l.at[rows, :], right, c)
            ccw = rc(qx.at[rows, :], gx_r.at[rows, :], left, 4 + c)
            cw.start()
            ccw.start()
            hop1 += [cw, ccw]
            if c + 2 < 4:
                nxt = pltpu.make_async_copy(
                    x_ref.at[pl.ds((c + 2) * mc, mc), :], stx.at[slot],
                    stsems.at[slot])
                nxt.start()
                stage.append(nxt)
            if c == 0:
                stage_w.wait()
                qw[...] = stw[...].astype(jnp.float8_e4m3fn)
                w_cw = rc(qw, gw_l, right, 8)
                w_ccw = rc(qw, gw_r, left, 9)
                w_cw.start()
                w_ccw.start()
                hop1 += [w_cw, w_ccw]

        out_ref[...] = jnp.dot(
            qx[...], qw[...],
            preferred_element_type=jnp.float32).astype(jnp.bfloat16)

        for r in hop1:
            r.wait()

        offs = (0, kq, kh, kh + kq)
        hop2 = []
        for c, o in enumerate(offs):
            dev = right if c < 2 else left
            src_x, src_w = (gx_l, gw_l) if c < 2 else (gx_r, gw_r)
            hop2.append((
                rc(src_x.at[:, pl.ds(o, kq)], gx2.at[:, pl.ds(o, kq)],
                   dev, 10 + 2 * c),
                rc(src_w.at[pl.ds(o, kq), :], gw2.at[pl.ds(o, kq), :],
                   dev, 11 + 2 * c),
            ))
        for rx, rw in hop2:
            rx.start()
            rw.start()

        out_ref[...] += jnp.dot(
            gx_l[...], gw_l[...],
            preferred_element_type=jnp.float32).astype(jnp.bfloat16)
        out_ref[...] += jnp.dot(
            gx_r[...], gw_r[...],
            preferred_element_type=jnp.float32).astype(jnp.bfloat16)

        for c in (0, 2, 1, 3):
            rx, rw = hop2[c]
            rx.wait()
            rw.wait()
            o = offs[c]
            out_ref[...] += jnp.dot(
                gx2[:, pl.ds(o, kq)], gw2[pl.ds(o, kq), :],
                preferred_element_type=jnp.float32).astype(jnp.bfloat16)

        y = out_ref[...].astype(jnp.float32) * (sx_ref[0] * sw_ref[0])
        out_ref[...] = (y * jax.nn.sigmoid(y)).astype(jnp.bfloat16)

    out = pl.pallas_call(
        body,
        out_shape=jax.ShapeDtypeStruct((m, n), jnp.bfloat16),
        in_specs=[
            pl.BlockSpec(memory_space=pl.ANY),
            pl.BlockSpec(memory_space=pl.ANY),
            pl.BlockSpec(memory_space=pltpu.SMEM),
            pl.BlockSpec(memory_space=pltpu.SMEM),
        ],
        out_specs=pl.BlockSpec(memory_space=pltpu.VMEM),
        scratch_shapes=[
            pltpu.VMEM((m, k_per), jnp.float8_e4m3fn),
            pltpu.VMEM((k_per, n), jnp.float8_e4m3fn),
            pltpu.VMEM((2, mc, k_per), jnp.float32),
            pltpu.VMEM((k_per, n), jnp.float32),
            pltpu.VMEM((m, k_per), jnp.float8_e4m3fn),
            pltpu.VMEM((k_per, n), jnp.float8_e4m3fn),
            pltpu.VMEM((m, k_per), jnp.float8_e4m3fn),
            pltpu.VMEM((k_per, n), jnp.float8_e4m3fn),
            pltpu.VMEM((m, k_per), jnp.float8_e4m3fn),
            pltpu.VMEM((k_per, n), jnp.float8_e4m3fn),
            pltpu.SemaphoreType.DMA((18,)),
            pltpu.SemaphoreType.DMA((18,)),
            pltpu.SemaphoreType.DMA((3,)),
        ],
        compiler_params=pltpu.CompilerParams(
            collective_id=0, vmem_limit_bytes=100 * 1024 * 1024),
    )(x, w_mat, scale_x, scale_w)
    return out
